# Initial kernel scaffold; baseline (speedup 1.0000x reference)
import jax
import jax.numpy as jnp
from jax.experimental import pallas as pl


def kernel(inputs, weights):
    raise NotImplementedError("write your pallas kernel here")


# speedup vs baseline: 1.8790x; 1.8790x over previous
import functools

import jax
import jax.numpy as jnp
from jax import lax
from jax.experimental import pallas as pl
from jax.experimental.pallas import tpu as pltpu
from jax.experimental.pallas import tpu_sc as plsc

EMB_DIM = 64
CHUNK = 128
NBUF = 4


def _embed_gather(table_hbm, idx_hbm, out_hbm, idx_v, rows_v, gsems, nc):
    wid = lax.axis_index("s") * nc + lax.axis_index("c")
    g_total = idx_v.shape[0]

    pltpu.sync_copy(idx_hbm.at[wid], idx_v)

    def start_gather(g, b):
        pltpu.async_copy(table_hbm.at[idx_v.at[g]], rows_v[b], gsems[b])

    def wait_gather(b):
        pltpu.make_async_copy(table_hbm.at[idx_v.at[0]], rows_v[b],
                              gsems[b]).wait()

    for b in range(NBUF):
        start_gather(b, b)

    def step(t):
        for b in range(NBUF):
            g = t + b
            wait_gather(b)
            pltpu.sync_copy(rows_v[b], out_hbm.at[wid, g])

            @pl.when(g + NBUF < g_total)
            def _():
                start_gather(g + NBUF, b)

    pl.loop(0, g_total, step=NBUF)(step)


def kernel(inputs, weights):
    n_rows, n_cols = inputs.shape
    total = n_rows * n_cols

    mesh = plsc.VectorSubcoreMesh(core_axis_name="c", subcore_axis_name="s")
    nw = mesh.num_cores * mesh.num_subcores
    g_per_w = total // (nw * CHUNK)
    assert g_per_w * nw * CHUNK == total

    idx = inputs.astype(jnp.int32).reshape(nw, g_per_w, CHUNK)

    scratch = (
        [pltpu.VMEM((g_per_w, CHUNK), jnp.int32)]
        + [pltpu.VMEM((CHUNK, EMB_DIM), jnp.float32) for _ in range(NBUF)]
        + [pltpu.SemaphoreType.DMA for _ in range(NBUF)]
    )

    def body(table_hbm, idx_hbm, out_hbm, idx_v, *rest):
        rows_v = rest[:NBUF]
        gsems = rest[NBUF:]
        _embed_gather(table_hbm, idx_hbm, out_hbm, idx_v, rows_v, gsems,
                      mesh.num_cores)

    out = pl.kernel(
        body,
        out_type=jax.ShapeDtypeStruct((nw, g_per_w, CHUNK, EMB_DIM),
                                      jnp.float32),
        mesh=mesh,
        scratch_types=scratch,
        compiler_params=pltpu.CompilerParams(use_tc_tiling_on_sc=False),
    )(weights, idx)

    return out.reshape(n_rows, n_cols, EMB_DIM)
